# baseline (device time: 66871 ns/iter reference)
import jax
import jax.numpy as jnp
from jax import lax
from jax.experimental import pallas as pl
from jax.experimental.pallas import tpu as pltpu

N_DEV = 4
B, SQ, SKV = 2, 512, 512
H_PER = 8
HQ = 32
DH = 64
D_MODEL = 768
D_HID = H_PER * DH
ROWS = B * SQ
CHUNK = ROWS // N_DEV
BF = jnp.bfloat16


def kernel(x, Wq, K_ext, V_ext, Wo):
    def body(x_ref, wq_ref, k_ext_ref, v_ext_ref, wo_ref, out_ref,
             ctx_scr, k_scr, v_scr,
             rs_send, rs_recv, ag_send, ag_recv, kv_sem,
             rs_send_sems, rs_recv_sems, ag_send_sems, ag_recv_sems):
        my_pos = lax.axis_index("i")
        h0 = my_pos * H_PER

        kv_copies = []
        for b in range(B):
            for h in range(H_PER):
                for src, dst in ((k_ext_ref, k_scr), (v_ext_ref, v_scr)):
                    c = pltpu.make_async_copy(
                        src.at[b, :, h0 + h, :], dst.at[b, h], kv_sem)
                    c.start()
                    kv_copies.append(c)

        barrier_sem = pltpu.get_barrier_semaphore()
        for k in range(1, N_DEV):
            pl.semaphore_signal(
                barrier_sem, inc=1,
                device_id=(lax.rem(my_pos + k, N_DEV),),
                device_id_type=pl.DeviceIdType.MESH)
        pl.semaphore_wait(barrier_sem, N_DEV - 1)

        qblk = lax.broadcasted_iota(jnp.int32, (SQ, SKV), 0) // 64
        kblk = lax.broadcasted_iota(jnp.int32, (SQ, SKV), 1) // 64
        mask = (qblk == kblk) | (kblk == 0) | (((qblk + kblk) % 3) == 0)
        bias = jnp.where(mask, 0.0, -1e9).astype(jnp.float32)

        wq_bf = wq_ref[...].astype(BF)
        wo_bf = wo_ref[...].astype(BF)

        waited = []

        for b in range(B):
            qb = jnp.dot(x_ref[b].astype(BF), wq_bf,
                         preferred_element_type=jnp.float32)
            qb = (qb * 0.125).astype(BF)
            if not waited:
                for c in kv_copies:
                    c.wait()
                waited.append(True)
            for h in range(H_PER):
                s = lax.dot_general(
                    qb[:, h * DH:(h + 1) * DH], k_scr[b, h].astype(BF),
                    (((1,), (1,)), ((), ())),
                    preferred_element_type=jnp.float32,
                )
                w = jnp.exp(s + bias)
                rinv = 1.0 / jnp.sum(w, axis=-1, keepdims=True)
                ctx_scr[:, h * DH:(h + 1) * DH] = jnp.dot(
                    w.astype(BF), v_scr[b, h].astype(BF),
                    preferred_element_type=jnp.float32) * rinv
            pb = jnp.dot(ctx_scr[...].astype(BF), wo_bf,
                         preferred_element_type=jnp.float32)
            out_ref[2 * b] = pb[:CHUNK]
            out_ref[2 * b + 1] = pb[CHUNK:]

            def rs_sends(P, b=b):
                for c in (2 * b, 2 * b + 1):
                    if c == P:
                        continue
                    rs_send[c] = out_ref[c].astype(BF)
                    pltpu.make_async_remote_copy(
                        src_ref=rs_send.at[c],
                        dst_ref=rs_recv.at[(P - c - 1) % N_DEV],
                        send_sem=rs_send_sems.at[c],
                        recv_sem=rs_recv_sems.at[(P - c - 1) % N_DEV],
                        device_id=(c,),
                        device_id_type=pl.DeviceIdType.MESH).start()
            for P in range(N_DEV):
                pl.when(my_pos == P)(lambda P=P: rs_sends(P))

        def reduce_and_ag(P):
            for s in range(N_DEV - 1):
                pltpu.make_async_remote_copy(
                    src_ref=rs_recv.at[s], dst_ref=rs_recv.at[s],
                    send_sem=rs_recv_sems.at[s],
                    recv_sem=rs_recv_sems.at[s],
                    device_id=(P,),
                    device_id_type=pl.DeviceIdType.MESH).wait_recv()
            acc = (out_ref[P]
                   + rs_recv[0].astype(jnp.float32)
                   + rs_recv[1].astype(jnp.float32)
                   + rs_recv[2].astype(jnp.float32))
            out_ref[P] = acc

            ag_send[...] = acc.astype(BF)
            ag_rdmas = []
            for k in range(N_DEV - 1):
                t = (P + 1 + k) % N_DEV
                r = pltpu.make_async_remote_copy(
                    src_ref=ag_send,
                    dst_ref=ag_recv.at[2 - k],
                    send_sem=ag_send_sems.at[k],
                    recv_sem=ag_recv_sems.at[2 - k],
                    device_id=(t,),
                    device_id_type=pl.DeviceIdType.MESH)
                r.start()
                ag_rdmas.append(r)

            for s in range(N_DEV - 1):
                pltpu.make_async_remote_copy(
                    src_ref=ag_recv.at[s], dst_ref=ag_recv.at[s],
                    send_sem=ag_send_sems.at[0],
                    recv_sem=ag_recv_sems.at[s],
                    device_id=(P,),
                    device_id_type=pl.DeviceIdType.MESH).wait_recv()
                out_ref[(P + 1 + s) % N_DEV] = ag_recv[s].astype(jnp.float32)

            for c in range(N_DEV):
                if c == P:
                    continue
                pltpu.make_async_remote_copy(
                    src_ref=rs_send.at[c], dst_ref=rs_send.at[c],
                    send_sem=rs_send_sems.at[c],
                    recv_sem=rs_recv_sems.at[0],
                    device_id=(P,),
                    device_id_type=pl.DeviceIdType.MESH).wait_send()
            for r in ag_rdmas:
                r.wait_send()

        for P in range(N_DEV):
            pl.when(my_pos == P)(lambda P=P: reduce_and_ag(P))

    out = pl.pallas_call(
        body,
        out_shape=jax.ShapeDtypeStruct((N_DEV, CHUNK, D_MODEL), jnp.float32),
        in_specs=[
            pl.BlockSpec(memory_space=pltpu.VMEM),
            pl.BlockSpec(memory_space=pltpu.VMEM),
            pl.BlockSpec(memory_space=pl.ANY),
            pl.BlockSpec(memory_space=pl.ANY),
            pl.BlockSpec(memory_space=pltpu.VMEM),
        ],
        out_specs=pl.BlockSpec(memory_space=pltpu.VMEM),
        scratch_shapes=[
            pltpu.VMEM((SQ, D_HID), jnp.float32),
            pltpu.VMEM((B, H_PER, SKV, DH), jnp.float32),
            pltpu.VMEM((B, H_PER, SKV, DH), jnp.float32),
            pltpu.VMEM((N_DEV, CHUNK, D_MODEL), BF),
            pltpu.VMEM((N_DEV - 1, CHUNK, D_MODEL), BF),
            pltpu.VMEM((CHUNK, D_MODEL), BF),
            pltpu.VMEM((N_DEV - 1, CHUNK, D_MODEL), BF),
            pltpu.SemaphoreType.DMA,
            pltpu.SemaphoreType.DMA((N_DEV,)),
            pltpu.SemaphoreType.DMA((N_DEV - 1,)),
            pltpu.SemaphoreType.DMA((N_DEV - 1,)),
            pltpu.SemaphoreType.DMA((N_DEV - 1,)),
        ],
        compiler_params=pltpu.CompilerParams(collective_id=0),
    )(x, Wq, K_ext, V_ext, Wo)
    return out.reshape(B, SQ, D_MODEL)


# device time: 38324 ns/iter; 1.7449x vs baseline; 1.7449x over previous
import jax
import jax.numpy as jnp
from jax import lax
from jax.experimental import pallas as pl
from jax.experimental.pallas import tpu as pltpu

N_DEV = 4
B, SQ, SKV = 2, 512, 512
H_PER = 8
HQ = 32
DH = 64
D_MODEL = 768
D_HID = H_PER * DH
ROWS = B * SQ
CHUNK = ROWS // N_DEV
BF = jnp.bfloat16


def kernel(x, Wq, K_ext, V_ext, Wo):
    i = lax.axis_index("i")
    K = lax.dynamic_slice_in_dim(
        K_ext.reshape(B, SKV, HQ * DH), i * D_HID, D_HID, axis=2).astype(BF)
    V = lax.dynamic_slice_in_dim(
        V_ext.reshape(B, SKV, HQ * DH), i * D_HID, D_HID, axis=2).astype(BF)

    def body(x_ref, wq_ref, k_ref, v_ref, wo_ref, out_ref, ctx_scr,
             rs_send, rs_recv, ag_send, ag_recv,
             rs_send_sems, rs_recv_sems, ag_send_sems, ag_recv_sems):
        my_pos = lax.axis_index("i")

        barrier_sem = pltpu.get_barrier_semaphore()
        for k in range(1, N_DEV):
            pl.semaphore_signal(
                barrier_sem, inc=1,
                device_id=(lax.rem(my_pos + k, N_DEV),),
                device_id_type=pl.DeviceIdType.MESH)
        pl.semaphore_wait(barrier_sem, N_DEV - 1)

        qblk = lax.broadcasted_iota(jnp.int32, (SQ, SKV), 0) // 64
        kblk = lax.broadcasted_iota(jnp.int32, (SQ, SKV), 1) // 64
        mask = (qblk == kblk) | (kblk == 0) | (((qblk + kblk) % 3) == 0)
        bias = jnp.where(mask, 0.0, -1e9).astype(jnp.float32)

        wq_bf = wq_ref[...].astype(BF)
        wo_bf = wo_ref[...].astype(BF)

        for b in range(B):
            qb = jnp.dot(x_ref[b].astype(BF), wq_bf,
                         preferred_element_type=jnp.float32)
            qb = (qb * 0.125).astype(BF)
            for h in range(H_PER):
                s = lax.dot_general(
                    qb[:, h * DH:(h + 1) * DH],
                    k_ref[b][:, h * DH:(h + 1) * DH],
                    (((1,), (1,)), ((), ())),
                    preferred_element_type=jnp.float32,
                )
                w = jnp.exp(s + bias)
                rinv = 1.0 / jnp.sum(w, axis=-1, keepdims=True)
                ctx_scr[:, h * DH:(h + 1) * DH] = jnp.dot(
                    w.astype(BF), v_ref[b][:, h * DH:(h + 1) * DH],
                    preferred_element_type=jnp.float32) * rinv
            pb = jnp.dot(ctx_scr[...].astype(BF), wo_bf,
                         preferred_element_type=jnp.float32)
            out_ref[2 * b] = pb[:CHUNK]
            out_ref[2 * b + 1] = pb[CHUNK:]

            def rs_sends(P, b=b):
                for c in (2 * b, 2 * b + 1):
                    if c == P:
                        continue
                    rs_send[c] = out_ref[c].astype(BF)
                    r = pltpu.make_async_remote_copy(
                        src_ref=rs_send.at[c],
                        dst_ref=rs_recv.at[(P - c - 1) % N_DEV],
                        send_sem=rs_send_sems.at[c],
                        recv_sem=rs_recv_sems.at[(P - c - 1) % N_DEV],
                        device_id=(c,),
                        device_id_type=pl.DeviceIdType.MESH)
                    r.start()
            for P in range(N_DEV):
                pl.when(my_pos == P)(lambda P=P: rs_sends(P))

        def reduce_and_ag(P):
            for s in range(N_DEV - 1):
                pltpu.make_async_remote_copy(
                    src_ref=rs_recv.at[s], dst_ref=rs_recv.at[s],
                    send_sem=rs_recv_sems.at[s],
                    recv_sem=rs_recv_sems.at[s],
                    device_id=(P,),
                    device_id_type=pl.DeviceIdType.MESH).wait_recv()
            out_ref[P] = (out_ref[P]
                          + rs_recv[0].astype(jnp.float32)
                          + rs_recv[1].astype(jnp.float32)
                          + rs_recv[2].astype(jnp.float32))

            ag_send[...] = out_ref[P].astype(BF)
            ag_rdmas = []
            for k in range(N_DEV - 1):
                t = (P + 1 + k) % N_DEV
                r = pltpu.make_async_remote_copy(
                    src_ref=ag_send,
                    dst_ref=ag_recv.at[2 - k],
                    send_sem=ag_send_sems.at[k],
                    recv_sem=ag_recv_sems.at[2 - k],
                    device_id=(t,),
                    device_id_type=pl.DeviceIdType.MESH)
                r.start()
                ag_rdmas.append(r)

            for s in range(N_DEV - 1):
                pltpu.make_async_remote_copy(
                    src_ref=ag_recv.at[s], dst_ref=ag_recv.at[s],
                    send_sem=ag_send_sems.at[0],
                    recv_sem=ag_recv_sems.at[s],
                    device_id=(P,),
                    device_id_type=pl.DeviceIdType.MESH).wait_recv()
                out_ref[(P + 1 + s) % N_DEV] = ag_recv[s].astype(jnp.float32)

            for c in range(N_DEV):
                if c == P:
                    continue
                pltpu.make_async_remote_copy(
                    src_ref=rs_send.at[c], dst_ref=rs_send.at[c],
                    send_sem=rs_send_sems.at[c],
                    recv_sem=rs_recv_sems.at[0],
                    device_id=(P,),
                    device_id_type=pl.DeviceIdType.MESH).wait_send()
            for r in ag_rdmas:
                r.wait_send()

        for P in range(N_DEV):
            pl.when(my_pos == P)(lambda P=P: reduce_and_ag(P))

    out = pl.pallas_call(
        body,
        out_shape=jax.ShapeDtypeStruct((N_DEV, CHUNK, D_MODEL), jnp.float32),
        in_specs=[pl.BlockSpec(memory_space=pltpu.VMEM)] * 5,
        out_specs=pl.BlockSpec(memory_space=pltpu.VMEM),
        scratch_shapes=[
            pltpu.VMEM((SQ, D_HID), jnp.float32),
            pltpu.VMEM((N_DEV, CHUNK, D_MODEL), BF),
            pltpu.VMEM((N_DEV - 1, CHUNK, D_MODEL), BF),
            pltpu.VMEM((CHUNK, D_MODEL), BF),
            pltpu.VMEM((N_DEV - 1, CHUNK, D_MODEL), BF),
            pltpu.SemaphoreType.DMA((N_DEV,)),
            pltpu.SemaphoreType.DMA((N_DEV - 1,)),
            pltpu.SemaphoreType.DMA((N_DEV - 1,)),
            pltpu.SemaphoreType.DMA((N_DEV - 1,)),
        ],
        compiler_params=pltpu.CompilerParams(collective_id=0),
    )(x, Wq, K, V, Wo)
    return out.reshape(B, SQ, D_MODEL)


# device time: 36446 ns/iter; 1.8348x vs baseline; 1.0515x over previous
import jax
import jax.numpy as jnp
from jax import lax
from jax.experimental import pallas as pl
from jax.experimental.pallas import tpu as pltpu

N_DEV = 4
B, SQ, SKV = 2, 512, 512
H_PER = 8
HQ = 32
DH = 64
D_MODEL = 768
D_HID = H_PER * DH
ROWS = B * SQ
NCH = 8
CH = ROWS // NCH
SUB = 256
BF = jnp.bfloat16


def kernel(x, Wq, K_ext, V_ext, Wo):
    i = lax.axis_index("i")
    K = lax.dynamic_slice_in_dim(
        K_ext.reshape(B, SKV, HQ * DH), i * D_HID, D_HID, axis=2).astype(BF)
    V = lax.dynamic_slice_in_dim(
        V_ext.reshape(B, SKV, HQ * DH), i * D_HID, D_HID, axis=2).astype(BF)

    def body(x_ref, wq_ref, k_ref, v_ref, wo_ref, out_ref, ctx_scr,
             rs_send, rs_recv, ag_send, ag_recv,
             rs_send_sems, rs_recv_sems, ag_send_sems, ag_recv_sems):
        my_pos = lax.axis_index("i")

        barrier_sem = pltpu.get_barrier_semaphore()
        for k in range(1, N_DEV):
            pl.semaphore_signal(
                barrier_sem, inc=1,
                device_id=(lax.rem(my_pos + k, N_DEV),),
                device_id_type=pl.DeviceIdType.MESH)
        pl.semaphore_wait(barrier_sem, N_DEV - 1)

        wq_bf = wq_ref[...].astype(BF)
        wo_bf = wo_ref[...].astype(BF)

        def sub_bias(half):
            qblk = (lax.broadcasted_iota(jnp.int32, (SUB, SKV), 0)
                    + half * SUB) // 64
            kblk = lax.broadcasted_iota(jnp.int32, (SUB, SKV), 1) // 64
            keep = (qblk == kblk) | (kblk == 0) | (((qblk + kblk) % 3) == 0)
            return jnp.where(keep, 0.0, -1e9).astype(jnp.float32)

        biases = [sub_bias(0), sub_bias(1)]

        def rs_sends(P, t):
            for c in (2 * t, 2 * t + 1):
                o = c % N_DEV
                if o == P:
                    continue
                rs_send[c] = out_ref[c].astype(BF)
                pltpu.make_async_remote_copy(
                    src_ref=rs_send.at[c],
                    dst_ref=rs_recv.at[(P - o - 1) % N_DEV + 3 * (c // 4)],
                    send_sem=rs_send_sems.at[c],
                    recv_sem=rs_recv_sems.at[(P - o - 1) % N_DEV + 3 * (c // 4)],
                    device_id=(o,),
                    device_id_type=pl.DeviceIdType.MESH).start()

        def reduce_and_ag(P, g):
            c = P + 4 * g
            for s in range(3):
                pltpu.make_async_remote_copy(
                    src_ref=rs_recv.at[3 * g + s],
                    dst_ref=rs_recv.at[3 * g + s],
                    send_sem=rs_recv_sems.at[3 * g + s],
                    recv_sem=rs_recv_sems.at[3 * g + s],
                    device_id=(P,),
                    device_id_type=pl.DeviceIdType.MESH).wait_recv()
            acc = (out_ref[c]
                   + rs_recv[3 * g].astype(jnp.float32)
                   + rs_recv[3 * g + 1].astype(jnp.float32)
                   + rs_recv[3 * g + 2].astype(jnp.float32))
            out_ref[c] = acc
            ag_send[g] = acc.astype(BF)
            for j in range(3):
                t = (P + 1 + j) % N_DEV
                pltpu.make_async_remote_copy(
                    src_ref=ag_send.at[g],
                    dst_ref=ag_recv.at[(P - t - 1) % N_DEV + 3 * g],
                    send_sem=ag_send_sems.at[3 * g + j],
                    recv_sem=ag_recv_sems.at[(P - t - 1) % N_DEV + 3 * g],
                    device_id=(t,),
                    device_id_type=pl.DeviceIdType.MESH).start()

        for t in range(4):
            b, half = divmod(t, 2)
            xq = x_ref[b, half * SUB:(half + 1) * SUB, :]
            qq = jnp.dot(xq.astype(BF), wq_bf,
                         preferred_element_type=jnp.float32)
            qq = (qq * 0.125).astype(BF)
            for h in range(H_PER):
                s = lax.dot_general(
                    qq[:, h * DH:(h + 1) * DH],
                    k_ref[b][:, h * DH:(h + 1) * DH],
                    (((1,), (1,)), ((), ())),
                    preferred_element_type=jnp.float32,
                )
                w = jnp.exp(s + biases[half])
                rinv = 1.0 / jnp.sum(w, axis=-1, keepdims=True)
                ctx_scr[:, h * DH:(h + 1) * DH] = jnp.dot(
                    w.astype(BF), v_ref[b][:, h * DH:(h + 1) * DH],
                    preferred_element_type=jnp.float32) * rinv
            pb = jnp.dot(ctx_scr[...].astype(BF), wo_bf,
                         preferred_element_type=jnp.float32)
            out_ref[2 * t] = pb[:CH]
            out_ref[2 * t + 1] = pb[CH:]
            for P in range(N_DEV):
                pl.when(my_pos == P)(lambda P=P, t=t: rs_sends(P, t))
            if t == 2:
                for P in range(N_DEV):
                    pl.when(my_pos == P)(lambda P=P: reduce_and_ag(P, 0))

        for P in range(N_DEV):
            pl.when(my_pos == P)(lambda P=P: reduce_and_ag(P, 1))

        def finish(P):
            for s in range(6):
                g, j = divmod(s, 3)
                o = (P + 1 + j) % N_DEV
                pltpu.make_async_remote_copy(
                    src_ref=ag_recv.at[s], dst_ref=ag_recv.at[s],
                    send_sem=ag_send_sems.at[0],
                    recv_sem=ag_recv_sems.at[s],
                    device_id=(P,),
                    device_id_type=pl.DeviceIdType.MESH).wait_recv()
                out_ref[o + 4 * g] = ag_recv[s].astype(jnp.float32)

            for c in range(NCH):
                if c % N_DEV == P:
                    continue
                pltpu.make_async_remote_copy(
                    src_ref=rs_send.at[c], dst_ref=rs_send.at[c],
                    send_sem=rs_send_sems.at[c],
                    recv_sem=rs_recv_sems.at[0],
                    device_id=(P,),
                    device_id_type=pl.DeviceIdType.MESH).wait_send()
            for s in range(6):
                pltpu.make_async_remote_copy(
                    src_ref=ag_send.at[s // 3], dst_ref=ag_send.at[s // 3],
                    send_sem=ag_send_sems.at[s],
                    recv_sem=ag_recv_sems.at[0],
                    device_id=(P,),
                    device_id_type=pl.DeviceIdType.MESH).wait_send()

        for P in range(N_DEV):
            pl.when(my_pos == P)(lambda P=P: finish(P))

    out = pl.pallas_call(
        body,
        out_shape=jax.ShapeDtypeStruct((NCH, CH, D_MODEL), jnp.float32),
        in_specs=[pl.BlockSpec(memory_space=pltpu.VMEM)] * 5,
        out_specs=pl.BlockSpec(memory_space=pltpu.VMEM),
        scratch_shapes=[
            pltpu.VMEM((SUB, D_HID), jnp.float32),
            pltpu.VMEM((NCH, CH, D_MODEL), BF),
            pltpu.VMEM((6, CH, D_MODEL), BF),
            pltpu.VMEM((2, CH, D_MODEL), BF),
            pltpu.VMEM((6, CH, D_MODEL), BF),
            pltpu.SemaphoreType.DMA((NCH,)),
            pltpu.SemaphoreType.DMA((6,)),
            pltpu.SemaphoreType.DMA((6,)),
            pltpu.SemaphoreType.DMA((6,)),
        ],
        compiler_params=pltpu.CompilerParams(collective_id=0),
    )(x, Wq, K, V, Wo)
    return out.reshape(B, SQ, D_MODEL)
